# baseline (speedup 1.0000x reference)
import functools

import jax
import jax.numpy as jnp
from jax import lax
from jax.experimental import pallas as pl
from jax.experimental.pallas import tpu as pltpu
from jax.experimental.pallas import tpu_sc as plsc

_N = 16384
_D = 1024
_NC = 2
_NS = 16
_NW = _NC * _NS
_ROWS_W = _N // _NW
_CHUNK = 32
_NCHUNK = _ROWS_W // _CHUNK
_NBUF = 3

_mesh = plsc.VectorSubcoreMesh(core_axis_name="c", subcore_axis_name="s")


@functools.partial(
    pl.kernel,
    out_type=jax.ShapeDtypeStruct((_N, _D), jnp.float32),
    mesh=_mesh,
    scratch_types=[
        *[pltpu.VMEM((_CHUNK, _D), jnp.float32) for _ in range(_NBUF)],
        *[pltpu.SemaphoreType.DMA for _ in range(2 * _NBUF)],
    ],
)
def _sc_copy(in_hbm, out_hbm, *scratch):
    bufs = scratch[:_NBUF]
    isems = scratch[_NBUF:2 * _NBUF]
    osems = scratch[2 * _NBUF:]
    wid = lax.axis_index("s") * _NC + lax.axis_index("c")
    base = wid * _ROWS_W

    def in_cp(i, b):
        return pltpu.make_async_copy(
            in_hbm.at[pl.ds(base + i * _CHUNK, _CHUNK), :], bufs[b], isems[b])

    def out_cp(i, b):
        return pltpu.make_async_copy(
            bufs[b], out_hbm.at[pl.ds(base + i * _CHUNK, _CHUNK), :], osems[b])

    for b in range(min(_NBUF, _NCHUNK)):
        in_cp(b, b).start()
    for i in range(_NCHUNK):
        b = i % _NBUF
        in_cp(i, b).wait()
        out_cp(i, b).start()
        if i + _NBUF < _NCHUNK:
            out_cp(i, b).wait()
            in_cp(i + _NBUF, b).start()
    for i in range(max(0, _NCHUNK - _NBUF), _NCHUNK):
        out_cp(i, i % _NBUF).wait()


def kernel(input):
    return _sc_copy(input)


# speedup vs baseline: 1.4751x; 1.4751x over previous
import jax
import jax.numpy as jnp
from jax.experimental import pallas as pl
from jax.experimental.pallas import tpu as pltpu

_N = 16384
_D = 1024
_CHUNK = 512
_NCHUNK = _N // _CHUNK
_NBUF = 8


def _copy_body(in_hbm, out_hbm, *scratch):
    bufs = scratch[:_NBUF]
    isems = scratch[_NBUF:2 * _NBUF]
    osems = scratch[2 * _NBUF:]

    def in_cp(i, b):
        return pltpu.make_async_copy(
            in_hbm.at[pl.ds(i * _CHUNK, _CHUNK), :], bufs[b], isems[b])

    def out_cp(i, b):
        return pltpu.make_async_copy(
            bufs[b], out_hbm.at[pl.ds(i * _CHUNK, _CHUNK), :], osems[b])

    for b in range(_NBUF):
        in_cp(b, b).start()
    for i in range(_NCHUNK):
        b = i % _NBUF
        in_cp(i, b).wait()
        out_cp(i, b).start()
        if i + _NBUF < _NCHUNK:
            out_cp(i, b).wait()
            in_cp(i + _NBUF, b).start()
    for i in range(_NCHUNK - _NBUF, _NCHUNK):
        out_cp(i, i % _NBUF).wait()


def kernel(input):
    return pl.pallas_call(
        _copy_body,
        in_specs=[pl.BlockSpec(memory_space=pltpu.MemorySpace.HBM)],
        out_specs=pl.BlockSpec(memory_space=pltpu.MemorySpace.HBM),
        out_shape=jax.ShapeDtypeStruct((_N, _D), jnp.float32),
        scratch_shapes=(
            [pltpu.VMEM((_CHUNK, _D), jnp.float32) for _ in range(_NBUF)]
            + [pltpu.SemaphoreType.DMA for _ in range(2 * _NBUF)]
        ),
    )(input)
